# baseline (speedup 1.0000x reference)
import functools

import jax
import jax.numpy as jnp
from jax import lax
from jax.experimental import pallas as pl
from jax.experimental.pallas import tpu as pltpu
from jax.experimental.pallas import tpu_sc as plsc

EMB = 16
NCORES = 2
NSUB = 16
NW = NCORES * NSUB
CHUNK = 2048
SUBIDX = 128
KSUB = CHUNK // SUBIDX
GROUPS = CHUNK // 16


def _renorm_chunk(rows_v):
    lane = lax.iota(jnp.int32, 16)

    @plsc.parallel_loop(0, GROUPS, unroll=2)
    def g_body(g):
        rowv = lane + g * 16
        cols = [
            plsc.load_gather(rows_v, [rowv, jnp.full((16,), j, jnp.int32)])
            for j in range(EMB)
        ]
        ss = cols[0] * cols[0]
        for j in range(1, EMB):
            ss = ss + cols[j] * cols[j]
        s = jnp.maximum(ss, 1.0)
        i = plsc.bitcast(s, jnp.int32)
        i = jnp.full((16,), 0x5F3759DF, jnp.int32) - lax.shift_right_logical(i, 1)
        y = plsc.bitcast(i, jnp.float32)
        for _ in range(3):
            y = y * (1.5 - 0.5 * s * y * y)
        scale = jnp.where(ss > 1.0, y, 1.0)
        for j in range(EMB):
            plsc.store_scatter(
                rows_v, [rowv, jnp.full((16,), j, jnp.int32)], cols[j] * scale
            )


def kernel(indices, node_emb):
    B, H = indices.shape
    N = B * H
    per_w = N // NW
    n_chunks = per_w // CHUNK
    idx2d = indices.reshape(N // SUBIDX, SUBIDX).astype(jnp.int32)

    mesh = plsc.VectorSubcoreMesh(core_axis_name="c", subcore_axis_name="s")

    @functools.partial(
        pl.kernel,
        mesh=mesh,
        out_type=jax.ShapeDtypeStruct((N, EMB), jnp.float32),
        compiler_params=pltpu.CompilerParams(
            needs_layout_passes=False, use_tc_tiling_on_sc=False
        ),
        scratch_types=[
            pltpu.VMEM((KSUB, SUBIDX), jnp.int32),
            pltpu.VMEM((KSUB, SUBIDX), jnp.int32),
            pltpu.VMEM((CHUNK, EMB), jnp.float32),
            pltpu.VMEM((CHUNK, EMB), jnp.float32),
            pltpu.SemaphoreType.DMA,
            pltpu.SemaphoreType.DMA,
        ],
    )
    def k(idx_hbm, table_hbm, out_hbm, idx_a, idx_b, rows_a, rows_b, sem_a, sem_b):
        wid = lax.axis_index("s") * NCORES + lax.axis_index("c")
        wbase = wid * per_w

        def fire(t, idx_v, rows_v, sem):
            cbase = wbase + t * CHUNK
            crow = pl.multiple_of(cbase // SUBIDX, 8)
            pltpu.sync_copy(idx_hbm.at[pl.ds(crow, KSUB)], idx_v)
            for j in range(KSUB):
                pltpu.async_copy(
                    table_hbm.at[idx_v.at[j]],
                    rows_v.at[pl.ds(j * SUBIDX, SUBIDX)],
                    sem,
                )

        def drain(rows_v, sem):
            for j in range(KSUB):
                pltpu.make_async_copy(
                    table_hbm.at[idx_a.at[0]],
                    rows_v.at[pl.ds(j * SUBIDX, SUBIDX)],
                    sem,
                ).wait()

        def finish(t, rows_v, sem):
            drain(rows_v, sem)
            _renorm_chunk(rows_v)
            cbase = wbase + t * CHUNK
            pltpu.sync_copy(rows_v, out_hbm.at[pl.ds(cbase, CHUNK)])

        fire(0, idx_a, rows_a, sem_a)

        def pair_body(g, carry):
            c0 = g * 2
            fire(c0 + 1, idx_b, rows_b, sem_b)
            finish(c0, rows_a, sem_a)

            @pl.when(c0 + 2 < n_chunks)
            def _():
                fire(c0 + 2, idx_a, rows_a, sem_a)

            finish(c0 + 1, rows_b, sem_b)
            return carry

        lax.fori_loop(0, n_chunks // 2, pair_body, 0)

    out = k(idx2d, node_emb)
    return out.reshape(B, H, EMB)


# speedup vs baseline: 1.0980x; 1.0980x over previous
import functools

import jax
import jax.numpy as jnp
from jax import lax
from jax.experimental import pallas as pl
from jax.experimental.pallas import tpu as pltpu
from jax.experimental.pallas import tpu_sc as plsc

EMB = 16
NCORES = 2
NSUB = 16
NW = NCORES * NSUB
CHUNK = 2048
SUBIDX = 128
KSUB = CHUNK // SUBIDX
GROUPS = CHUNK // 16


def _renorm_chunk(rows_v):
    lane = lax.iota(jnp.int32, 16)

    @plsc.parallel_loop(0, GROUPS)
    def g_body(g):
        rowv = lane + g * 16
        cols = [
            plsc.load_gather(rows_v, [rowv, jnp.full((16,), j, jnp.int32)])
            for j in range(EMB)
        ]
        ss = cols[0] * cols[0]
        for j in range(1, EMB):
            ss = ss + cols[j] * cols[j]
        s = jnp.maximum(ss, 1.0)
        i = plsc.bitcast(s, jnp.int32)
        i = jnp.full((16,), 0x5F3759DF, jnp.int32) - lax.shift_right_logical(i, 1)
        y = plsc.bitcast(i, jnp.float32)
        for _ in range(3):
            y = y * (1.5 - 0.5 * s * y * y)
        scale = jnp.where(ss > 1.0, y, 1.0)
        for j in range(EMB):
            plsc.store_scatter(
                rows_v, [rowv, jnp.full((16,), j, jnp.int32)], cols[j] * scale
            )


def kernel(indices, node_emb):
    B, H = indices.shape
    N = B * H
    per_w = N // NW
    n_chunks = per_w // CHUNK
    idx_flat = indices.reshape(N).astype(jnp.int32)

    mesh = plsc.VectorSubcoreMesh(core_axis_name="c", subcore_axis_name="s")

    @functools.partial(
        pl.kernel,
        mesh=mesh,
        out_type=jax.ShapeDtypeStruct((N, EMB), jnp.float32),
        compiler_params=pltpu.CompilerParams(
            needs_layout_passes=False, use_tc_tiling_on_sc=False
        ),
        scratch_types=[
            pltpu.VMEM((CHUNK,), jnp.int32),
            pltpu.VMEM((CHUNK,), jnp.int32),
            pltpu.VMEM((CHUNK, EMB), jnp.float32),
            pltpu.VMEM((CHUNK, EMB), jnp.float32),
            pltpu.SemaphoreType.DMA,
            pltpu.SemaphoreType.DMA,
        ],
    )
    def k(idx_hbm, table_hbm, out_hbm, idx_a, idx_b, rows_a, rows_b, sem_a, sem_b):
        wid = lax.axis_index("s") * NCORES + lax.axis_index("c")
        wbase = wid * per_w

        def fire(t, idx_v, rows_v, sem):
            cbase = wbase + t * CHUNK
            cb = pl.multiple_of(cbase, 8)
            pltpu.sync_copy(idx_hbm.at[pl.ds(cb, CHUNK)], idx_v)
            pltpu.async_copy(table_hbm.at[idx_v], rows_v, sem)

        def drain(rows_v, sem):
            pltpu.make_async_copy(table_hbm.at[idx_a], rows_v, sem).wait()

        def finish(t, rows_v, sem):
            drain(rows_v, sem)
            _renorm_chunk(rows_v)
            cbase = wbase + t * CHUNK
            pltpu.sync_copy(rows_v, out_hbm.at[pl.ds(cbase, CHUNK)])

        fire(0, idx_a, rows_a, sem_a)

        def pair_body(g, carry):
            c0 = g * 2
            fire(c0 + 1, idx_b, rows_b, sem_b)
            finish(c0, rows_a, sem_a)

            @pl.when(c0 + 2 < n_chunks)
            def _():
                fire(c0 + 2, idx_a, rows_a, sem_a)

            finish(c0 + 1, rows_b, sem_b)
            return carry

        lax.fori_loop(0, n_chunks // 2, pair_body, 0)

    out = k(idx_flat, node_emb)
    return out.reshape(B, H, EMB)


# speedup vs baseline: 1.1027x; 1.0043x over previous
import functools

import jax
import jax.numpy as jnp
from jax import lax
from jax.experimental import pallas as pl
from jax.experimental.pallas import tpu as pltpu
from jax.experimental.pallas import tpu_sc as plsc

EMB = 16
NCORES = 2
NSUB = 16
NW = NCORES * NSUB
CHUNK = 2048
NBUF = 3
GROUPS = CHUNK // 16


def _renorm_chunk(rows_v):
    lane = lax.iota(jnp.int32, 16)

    @plsc.parallel_loop(0, GROUPS)
    def g_body(g):
        rowv = lane + g * 16
        cols = [
            plsc.load_gather(rows_v, [rowv, jnp.full((16,), j, jnp.int32)])
            for j in range(EMB)
        ]
        ss = cols[0] * cols[0]
        for j in range(1, EMB):
            ss = ss + cols[j] * cols[j]
        s = jnp.maximum(ss, 1.0)
        i = plsc.bitcast(s, jnp.int32)
        i = jnp.full((16,), 0x5F3759DF, jnp.int32) - lax.shift_right_logical(i, 1)
        y = plsc.bitcast(i, jnp.float32)
        for _ in range(3):
            y = y * (1.5 - 0.5 * s * y * y)
        scale = jnp.where(ss > 1.0, y, 1.0)
        for j in range(EMB):
            plsc.store_scatter(
                rows_v, [rowv, jnp.full((16,), j, jnp.int32)], cols[j] * scale
            )


def kernel(indices, node_emb):
    B, H = indices.shape
    N = B * H
    per_w = N // NW
    n_chunks = per_w // CHUNK
    idx_flat = indices.reshape(N).astype(jnp.int32)

    mesh = plsc.VectorSubcoreMesh(core_axis_name="c", subcore_axis_name="s")

    @functools.partial(
        pl.kernel,
        mesh=mesh,
        out_type=jax.ShapeDtypeStruct((N, EMB), jnp.float32),
        compiler_params=pltpu.CompilerParams(
            needs_layout_passes=False, use_tc_tiling_on_sc=False
        ),
        scratch_types=[
            [pltpu.VMEM((CHUNK,), jnp.int32) for _ in range(NBUF)],
            [pltpu.VMEM((CHUNK, EMB), jnp.float32) for _ in range(NBUF)],
            [pltpu.SemaphoreType.DMA for _ in range(NBUF)],
            [pltpu.SemaphoreType.DMA for _ in range(NBUF)],
        ],
    )
    def k(idx_hbm, table_hbm, out_hbm, idx_bufs, row_bufs, gsems, wsems):
        wid = lax.axis_index("s") * NCORES + lax.axis_index("c")
        wbase = wid * per_w

        def fire(c, b):
            @pl.when(c >= NBUF)
            def _():
                pltpu.make_async_copy(
                    row_bufs[b], out_hbm.at[pl.ds(0, CHUNK)], wsems[b]
                ).wait()

            cbase = wbase + c * CHUNK
            cb = pl.multiple_of(cbase, 8)
            pltpu.sync_copy(idx_hbm.at[pl.ds(cb, CHUNK)], idx_bufs[b])
            pltpu.async_copy(table_hbm.at[idx_bufs[b]], row_bufs[b], gsems[b])

        def finish(c, b):
            pltpu.make_async_copy(
                table_hbm.at[idx_bufs[b]], row_bufs[b], gsems[b]
            ).wait()
            _renorm_chunk(row_bufs[b])
            cbase = wbase + c * CHUNK
            pltpu.async_copy(
                row_bufs[b], out_hbm.at[pl.ds(cbase, CHUNK)], wsems[b]
            )

        def quad_body(q, carry):
            for b in range(NBUF):
                c = q * NBUF + b

                @pl.when(c < n_chunks)
                def _():
                    fire(c, b)

                cf = c - (NBUF - 1)
                fb = (b + 1) % NBUF

                @pl.when((cf >= 0) & (cf < n_chunks))
                def _():
                    finish(cf, fb)

            return carry

        n_steps = n_chunks + NBUF - 1
        lax.fori_loop(0, (n_steps + NBUF - 1) // NBUF, quad_body, 0)

        for b in range(NBUF):
            pltpu.make_async_copy(
                row_bufs[b], out_hbm.at[pl.ds(0, CHUNK)], wsems[b]
            ).wait()

    out = k(idx_flat, node_emb)
    return out.reshape(B, H, EMB)
